# baseline (speedup 1.0000x reference)
import functools

import jax
import jax.numpy as jnp
from jax.experimental import pallas as pl


def _conv2d(x, w, b, stride, pad):
    out = jax.lax.conv_general_dilated(
        x, w, window_strides=(stride, stride),
        padding=[(pad, pad), (pad, pad)],
        dimension_numbers=('NCHW', 'OIHW', 'NCHW'))
    return out + b[None, :, None, None]


def _vq_kernel(flat_ref, rn_ref, emb_ref, cn_ref, q_ref, sq_ref):
    i = pl.program_id(0)
    f = flat_ref[...]
    e = emb_ref[...]
    fe = jax.lax.dot_general(f, e, (((1,), (1,)), ((), ())),
                             preferred_element_type=jnp.float32)
    d2 = rn_ref[...] - 2.0 * fe + cn_ref[...]
    dist = jnp.sqrt(jnp.maximum(d2, 0.0))
    minv = jnp.min(dist, axis=1, keepdims=True)
    kiota = jax.lax.broadcasted_iota(jnp.int32, dist.shape, 1)
    big = jnp.int32(dist.shape[1])
    idx = jnp.min(jnp.where(dist == minv, kiota, big), axis=1)
    onehot = (idx[:, None] == kiota).astype(jnp.float32)
    q = jax.lax.dot_general(onehot, e, (((1,), (0,)), ((), ())),
                            preferred_element_type=jnp.float32)
    q_ref[...] = q
    diff = q - f
    part = jnp.sum(diff * diff).reshape(1, 1)

    @pl.when(i == 0)
    def _init():
        sq_ref[...] = part

    @pl.when(i != 0)
    def _acc():
        sq_ref[...] += part


@functools.partial(jax.jit, static_argnames=("blk",))
def _vq(flat, emb, blk=512):
    n, c = flat.shape
    k = emb.shape[0]
    rn = jnp.sum(flat ** 2, axis=1, keepdims=True)
    cn = jnp.sum(emb ** 2, axis=1)[None, :]
    grid = n // blk
    q, sq = pl.pallas_call(
        _vq_kernel,
        grid=(grid,),
        in_specs=[
            pl.BlockSpec((blk, c), lambda i: (i, 0)),
            pl.BlockSpec((blk, 1), lambda i: (i, 0)),
            pl.BlockSpec((k, c), lambda i: (0, 0)),
            pl.BlockSpec((1, k), lambda i: (0, 0)),
        ],
        out_specs=[
            pl.BlockSpec((blk, c), lambda i: (i, 0)),
            pl.BlockSpec((1, 1), lambda i: (0, 0)),
        ],
        out_shape=[
            jax.ShapeDtypeStruct((n, c), jnp.float32),
            jax.ShapeDtypeStruct((1, 1), jnp.float32),
        ],
    )(flat, rn, emb, cn)
    return q, sq[0, 0]


_KTAP = ((3, 1), (2, 0))


def _phase_weight(w):
    cin, cout = w.shape[0], w.shape[1]
    wb = jnp.zeros((9 * cin, 4 * cout), w.dtype)
    for rh in range(2):
        for rw in range(2):
            for dh in range(2):
                for dw_ in range(2):
                    eh, ew = rh + dh, rw + dw_
                    tap = w[:, :, _KTAP[rh][dh], _KTAP[rw][dw_]]
                    wb = wb.at[(eh * 3 + ew) * cin:(eh * 3 + ew + 1) * cin,
                               (rh * 2 + rw) * cout:(rh * 2 + rw + 1) * cout
                               ].set(tap)
    return wb


def _patchmm_kernel(xp_ref, w_ref, b_ref, out_ref, *, rb, wdim, cin, nout, act):
    s = pl.program_id(1)
    base = s * rb
    chunks = []
    for eh in range(3):
        for ew in range(3):
            sl = xp_ref[0, pl.ds(base + eh, rb), ew:ew + wdim, :]
            chunks.append(sl.reshape(rb * wdim, cin))
    p = jnp.concatenate(chunks, axis=1)
    res = jax.lax.dot_general(p, w_ref[...], (((1,), (0,)), ((), ())),
                              preferred_element_type=jnp.float32)
    res = res + b_ref[...]
    if act == "relu":
        res = jnp.maximum(res, 0.0)
    else:
        res = 1.0 / (1.0 + jnp.exp(-res))
    out_ref[0] = res[:, :nout].reshape(rb, wdim, nout)


@functools.partial(jax.jit, static_argnames=("rb", "nout", "act"))
def _patchmm(xh, wb, bb, rb, nout, act):
    bsz, h, wdim, cin = xh.shape
    nmat = wb.shape[1]
    xp = jnp.pad(xh, ((0, 0), (1, 1), (1, 1), (0, 0)))
    nstrip = h // rb
    body = functools.partial(_patchmm_kernel, rb=rb, wdim=wdim, cin=cin,
                             nout=nout, act=act)
    return pl.pallas_call(
        body,
        grid=(bsz, nstrip),
        in_specs=[
            pl.BlockSpec((1, h + 2, wdim + 2, cin), lambda i, s: (i, 0, 0, 0)),
            pl.BlockSpec((9 * cin, nmat), lambda i, s: (0, 0)),
            pl.BlockSpec((1, nmat), lambda i, s: (0, 0)),
        ],
        out_specs=pl.BlockSpec((1, rb, wdim, nout),
                               lambda i, s: (i, s, 0, 0)),
        out_shape=jax.ShapeDtypeStruct((bsz, h, wdim, nout), jnp.float32),
    )(xp, wb, bb)


def _deconv_phase(xh, w, b, rb):
    cout = w.shape[1]
    wb = _phase_weight(w)
    bb = jnp.tile(b, 4)[None, :]
    return _patchmm(xh, wb, bb, rb, 4 * cout, "relu")


def _interleave(ph, cout):
    bsz, h, wdim = ph.shape[:3]
    ph = ph.reshape(bsz, h, wdim, 2, 2, cout)
    ph = ph.transpose(0, 1, 3, 2, 4, 5)
    return ph.reshape(bsz, 2 * h, 2 * wdim, cout)


def _conv3_kernel(a0_ref, a1_ref, a2_ref, w_ref, b_ref, out_ref,
                  *, rb, wdim, cin, nout):
    refs = (a0_ref, a1_ref, a2_ref)
    chunks = []
    for eh in range(3):
        for ew in range(3):
            sl = refs[eh][0, :, ew:ew + wdim, :]
            chunks.append(sl.reshape(rb * wdim, cin))
    p = jnp.concatenate(chunks, axis=1)
    res = jax.lax.dot_general(p, w_ref[...], (((1,), (0,)), ((), ())),
                              preferred_element_type=jnp.float32)
    res = 1.0 / (1.0 + jnp.exp(-(res + b_ref[...])))
    out_ref[0] = res[:, :nout].reshape(rb, wdim, nout)


@functools.partial(jax.jit, static_argnames=("rb", "nout"))
def _conv3(xh, wb, bb, rb, nout):
    bsz, h, wdim, cin = xh.shape
    nmat = wb.shape[1]
    xp = jnp.pad(xh, ((0, 0), (1, 1), (1, 1), (0, 0)))
    shifted = [xp[:, eh:eh + h] for eh in range(3)]
    nstrip = h // rb
    body = functools.partial(_conv3_kernel, rb=rb, wdim=wdim, cin=cin,
                             nout=nout)
    inspec = pl.BlockSpec((1, rb, wdim + 2, cin), lambda i, s: (i, s, 0, 0))
    return pl.pallas_call(
        body,
        grid=(bsz, nstrip),
        in_specs=[
            inspec, inspec, inspec,
            pl.BlockSpec((9 * cin, nmat), lambda i, s: (0, 0)),
            pl.BlockSpec((1, nmat), lambda i, s: (0, 0)),
        ],
        out_specs=pl.BlockSpec((1, rb, wdim, nout),
                               lambda i, s: (i, s, 0, 0)),
        out_shape=jax.ShapeDtypeStruct((bsz, h, wdim, nout), jnp.float32),
    )(*shifted, wb, bb)


def kernel(x, w1, b1, w2, b2, emb, dw1, db1, dw2, db2, w3, b3):
    z = jax.nn.relu(_conv2d(x, w1, b1, 2, 1))
    z = jax.nn.relu(_conv2d(z, w2, b2, 2, 1))
    B, C, H, W = z.shape
    flat = z.transpose(0, 2, 3, 1).reshape(-1, C)
    q_flat, sq = _vq(flat, emb)
    loss = 1.25 * sq / (flat.shape[0] * C)
    q_nhwc = q_flat.reshape(B, H, W, C)

    h1 = _interleave(_deconv_phase(q_nhwc, dw1, db1, rb=56), dw1.shape[1])
    h2 = _interleave(_deconv_phase(h1, dw2, db2, rb=28), dw2.shape[1])

    cin3, co3 = w3.shape[1], w3.shape[0]
    w3m = w3.transpose(2, 3, 1, 0).reshape(9 * cin3, co3)
    w3p = jnp.zeros((9 * cin3, 128), w3.dtype).at[:, :co3].set(w3m)
    b3p = jnp.zeros((1, 128), b3.dtype).at[0, :co3].set(b3)
    recon_nhwc = _conv3(h2, w3p, b3p, rb=28, nout=co3)
    recon = recon_nhwc.transpose(0, 3, 1, 2)
    q_st = q_nhwc.transpose(0, 3, 1, 2)
    return (recon, loss, q_st)


# speedup vs baseline: 1.1514x; 1.1514x over previous
import functools

import jax
import jax.numpy as jnp
from jax.experimental import pallas as pl


def _conv2d(x, w, b, stride, pad):
    out = jax.lax.conv_general_dilated(
        x, w, window_strides=(stride, stride),
        padding=[(pad, pad), (pad, pad)],
        dimension_numbers=('NCHW', 'OIHW', 'NCHW'))
    return out + b[None, :, None, None]


def _vq_kernel(flat_ref, rn_ref, emb_ref, cn_ref, q_ref, sq_ref):
    i = pl.program_id(0)
    f = flat_ref[...]
    e = emb_ref[...]
    fe = jax.lax.dot_general(f, e, (((1,), (1,)), ((), ())),
                             preferred_element_type=jnp.float32)
    d2 = rn_ref[...] - 2.0 * fe + cn_ref[...]
    dist = jnp.sqrt(jnp.maximum(d2, 0.0))
    minv = jnp.min(dist, axis=1, keepdims=True)
    kiota = jax.lax.broadcasted_iota(jnp.int32, dist.shape, 1)
    big = jnp.int32(dist.shape[1])
    idx = jnp.min(jnp.where(dist == minv, kiota, big), axis=1)
    onehot = (idx[:, None] == kiota).astype(jnp.float32)
    q = jax.lax.dot_general(onehot, e, (((1,), (0,)), ((), ())),
                            preferred_element_type=jnp.float32)
    q_ref[...] = q
    diff = q - f
    part = jnp.sum(diff * diff).reshape(1, 1)

    @pl.when(i == 0)
    def _init():
        sq_ref[...] = part

    @pl.when(i != 0)
    def _acc():
        sq_ref[...] += part


@functools.partial(jax.jit, static_argnames=("blk",))
def _vq(flat, emb, blk=512):
    n, c = flat.shape
    k = emb.shape[0]
    rn = jnp.sum(flat ** 2, axis=1, keepdims=True)
    cn = jnp.sum(emb ** 2, axis=1)[None, :]
    grid = n // blk
    q, sq = pl.pallas_call(
        _vq_kernel,
        grid=(grid,),
        in_specs=[
            pl.BlockSpec((blk, c), lambda i: (i, 0)),
            pl.BlockSpec((blk, 1), lambda i: (i, 0)),
            pl.BlockSpec((k, c), lambda i: (0, 0)),
            pl.BlockSpec((1, k), lambda i: (0, 0)),
        ],
        out_specs=[
            pl.BlockSpec((blk, c), lambda i: (i, 0)),
            pl.BlockSpec((1, 1), lambda i: (0, 0)),
        ],
        out_shape=[
            jax.ShapeDtypeStruct((n, c), jnp.float32),
            jax.ShapeDtypeStruct((1, 1), jnp.float32),
        ],
    )(flat, rn, emb, cn)
    return q, sq[0, 0]


_KTAP = ((3, 1), (2, 0))


def _phase_weight(w):
    cin, cout = w.shape[0], w.shape[1]
    wb = jnp.zeros((9 * cin, 4 * cout), w.dtype)
    for rh in range(2):
        for rw in range(2):
            for dh in range(2):
                for dw_ in range(2):
                    eh, ew = rh + dh, rw + dw_
                    tap = w[:, :, _KTAP[rh][dh], _KTAP[rw][dw_]]
                    wb = wb.at[(eh * 3 + ew) * cin:(eh * 3 + ew + 1) * cin,
                               (rh * 2 + rw) * cout:(rh * 2 + rw + 1) * cout
                               ].set(tap)
    return wb


def _patchmm_kernel(xp_ref, w_ref, b_ref, out_ref, *, rb, wdim, cin, nout, act):
    s = pl.program_id(1)
    base = s * rb
    chunks = []
    for eh in range(3):
        for ew in range(3):
            sl = xp_ref[0, pl.ds(base + eh, rb), ew:ew + wdim, :]
            chunks.append(sl.reshape(rb * wdim, cin))
    p = jnp.concatenate(chunks, axis=1)
    res = jax.lax.dot_general(p, w_ref[...], (((1,), (0,)), ((), ())),
                              preferred_element_type=jnp.float32)
    res = res + b_ref[...]
    if act == "relu":
        res = jnp.maximum(res, 0.0)
    else:
        res = 1.0 / (1.0 + jnp.exp(-res))
    out_ref[0] = res[:, :nout].reshape(rb, wdim, nout)


@functools.partial(jax.jit, static_argnames=("rb", "nout", "act"))
def _patchmm(xh, wb, bb, rb, nout, act):
    bsz, h, wdim, cin = xh.shape
    nmat = wb.shape[1]
    xp = jnp.pad(xh, ((0, 0), (1, 1), (1, 1), (0, 0)))
    nstrip = h // rb
    body = functools.partial(_patchmm_kernel, rb=rb, wdim=wdim, cin=cin,
                             nout=nout, act=act)
    return pl.pallas_call(
        body,
        grid=(bsz, nstrip),
        in_specs=[
            pl.BlockSpec((1, h + 2, wdim + 2, cin), lambda i, s: (i, 0, 0, 0)),
            pl.BlockSpec((9 * cin, nmat), lambda i, s: (0, 0)),
            pl.BlockSpec((1, nmat), lambda i, s: (0, 0)),
        ],
        out_specs=pl.BlockSpec((1, rb, wdim, nout),
                               lambda i, s: (i, s, 0, 0)),
        out_shape=jax.ShapeDtypeStruct((bsz, h, wdim, nout), jnp.float32),
    )(xp, wb, bb)


def _deconv_phase(xh, w, b, rb):
    cout = w.shape[1]
    wb = _phase_weight(w)
    bb = jnp.tile(b, 4)[None, :]
    return _patchmm(xh, wb, bb, rb, 4 * cout, "relu")


def _interleave(ph, cout):
    bsz, h, wdim = ph.shape[:3]
    ph = ph.reshape(bsz, h, wdim, 2, 2, cout)
    ph = ph.transpose(0, 1, 3, 2, 4, 5)
    return ph.reshape(bsz, 2 * h, 2 * wdim, cout)


def _conv3_kernel(a0_ref, a1_ref, a2_ref, w_ref, b_ref, out_ref,
                  *, rb, nsc):
    refs = (a0_ref, a1_ref, a2_ref)
    chunks = []
    for eh in range(3):
        for rs in range(3):
            sl = refs[eh][0, :, rs:rs + nsc, :]
            chunks.append(sl.reshape(rb * nsc, 128))
    p = jnp.concatenate(chunks, axis=1)
    res = jax.lax.dot_general(p, w_ref[...], (((1,), (0,)), ((), ())),
                              preferred_element_type=jnp.float32)
    res = 1.0 / (1.0 + jnp.exp(-(res + b_ref[...])))
    out_ref[0] = res.reshape(rb, nsc, 128)


@functools.partial(jax.jit, static_argnames=("rb",))
def _conv3(xh, wb, bb, rb):
    bsz, h, wdim, cin = xh.shape
    nsc = wdim // 4
    xp = jnp.pad(xh, ((0, 0), (1, 1), (4, 4), (0, 0)))
    g = xp.reshape(bsz, h + 2, nsc + 2, 4 * cin)
    shifted = [g[:, eh:eh + h] for eh in range(3)]
    nstrip = h // rb
    body = functools.partial(_conv3_kernel, rb=rb, nsc=nsc)
    inspec = pl.BlockSpec((1, rb, nsc + 2, 128), lambda i, s: (i, s, 0, 0))
    return pl.pallas_call(
        body,
        grid=(bsz, nstrip),
        in_specs=[
            inspec, inspec, inspec,
            pl.BlockSpec((1152, 128), lambda i, s: (0, 0)),
            pl.BlockSpec((1, 128), lambda i, s: (0, 0)),
        ],
        out_specs=pl.BlockSpec((1, rb, nsc, 128), lambda i, s: (i, s, 0, 0)),
        out_shape=jax.ShapeDtypeStruct((bsz, h, nsc, 128), jnp.float32),
    )(*shifted, wb, bb)


def kernel(x, w1, b1, w2, b2, emb, dw1, db1, dw2, db2, w3, b3):
    z = jax.nn.relu(_conv2d(x, w1, b1, 2, 1))
    z = jax.nn.relu(_conv2d(z, w2, b2, 2, 1))
    B, C, H, W = z.shape
    flat = z.transpose(0, 2, 3, 1).reshape(-1, C)
    q_flat, sq = _vq(flat, emb)
    loss = 1.25 * sq / (flat.shape[0] * C)
    q_nhwc = q_flat.reshape(B, H, W, C)

    h1 = _interleave(_deconv_phase(q_nhwc, dw1, db1, rb=56), dw1.shape[1])
    h2 = _interleave(_deconv_phase(h1, dw2, db2, rb=28), dw2.shape[1])

    cin3, co3 = w3.shape[1], w3.shape[0]
    w3f = jnp.zeros((1152, 128), w3.dtype)
    for eh in range(3):
        for dw_ in range(3):
            tap = w3[:, :, eh, dw_].T
            for j4o in range(4):
                rs, j4i = (j4o + dw_ + 3) // 4, (j4o + dw_ + 3) % 4
                r0 = (eh * 3 + rs) * 128 + j4i * 32
                w3f = w3f.at[r0:r0 + cin3, j4o * 32:j4o * 32 + co3].set(tap)
    b3f = jnp.zeros((1, 128), b3.dtype)
    for j4o in range(4):
        b3f = b3f.at[0, j4o * 32:j4o * 32 + co3].set(b3)
    rec = _conv3(h2, w3f, b3f, rb=28)
    recon_nhwc = rec.reshape(8, 224, 224, 32)[..., :co3]
    recon = recon_nhwc.transpose(0, 3, 1, 2)
    q_st = q_nhwc.transpose(0, 3, 1, 2)
    return (recon, loss, q_st)


# speedup vs baseline: 1.1787x; 1.0237x over previous
import functools

import jax
import jax.numpy as jnp
from jax.experimental import pallas as pl


def _conv2d(x, w, b, stride, pad):
    out = jax.lax.conv_general_dilated(
        x, w, window_strides=(stride, stride),
        padding=[(pad, pad), (pad, pad)],
        dimension_numbers=('NCHW', 'OIHW', 'NCHW'))
    return out + b[None, :, None, None]


def _vq_kernel(flat_ref, rn_ref, emb_ref, cn_ref, q_ref, sq_ref):
    i = pl.program_id(0)
    f = flat_ref[...]
    e = emb_ref[...]
    fe = jax.lax.dot_general(f, e, (((1,), (1,)), ((), ())),
                             preferred_element_type=jnp.float32)
    d2 = rn_ref[...] - 2.0 * fe + cn_ref[...]
    dist = jnp.sqrt(jnp.maximum(d2, 0.0))
    minv = jnp.min(dist, axis=1, keepdims=True)
    kiota = jax.lax.broadcasted_iota(jnp.int32, dist.shape, 1)
    big = jnp.int32(dist.shape[1])
    idx = jnp.min(jnp.where(dist == minv, kiota, big), axis=1)
    onehot = (idx[:, None] == kiota).astype(jnp.float32)
    q = jax.lax.dot_general(onehot, e, (((1,), (0,)), ((), ())),
                            preferred_element_type=jnp.float32)
    q_ref[...] = q
    diff = q - f
    part = jnp.sum(diff * diff).reshape(1, 1)

    @pl.when(i == 0)
    def _init():
        sq_ref[...] = part

    @pl.when(i != 0)
    def _acc():
        sq_ref[...] += part


@functools.partial(jax.jit, static_argnames=("blk",))
def _vq(flat, emb, blk=512):
    n, c = flat.shape
    k = emb.shape[0]
    rn = jnp.sum(flat ** 2, axis=1, keepdims=True)
    cn = jnp.sum(emb ** 2, axis=1)[None, :]
    grid = n // blk
    q, sq = pl.pallas_call(
        _vq_kernel,
        grid=(grid,),
        in_specs=[
            pl.BlockSpec((blk, c), lambda i: (i, 0)),
            pl.BlockSpec((blk, 1), lambda i: (i, 0)),
            pl.BlockSpec((k, c), lambda i: (0, 0)),
            pl.BlockSpec((1, k), lambda i: (0, 0)),
        ],
        out_specs=[
            pl.BlockSpec((blk, c), lambda i: (i, 0)),
            pl.BlockSpec((1, 1), lambda i: (0, 0)),
        ],
        out_shape=[
            jax.ShapeDtypeStruct((n, c), jnp.float32),
            jax.ShapeDtypeStruct((1, 1), jnp.float32),
        ],
    )(flat, rn, emb, cn)
    return q, sq[0, 0]


_KTAP = ((3, 1), (2, 0))


def _phase_weight(w):
    cin, cout = w.shape[0], w.shape[1]
    wb = jnp.zeros((9 * cin, 4 * cout), w.dtype)
    for rh in range(2):
        for rw in range(2):
            for dh in range(2):
                for dw_ in range(2):
                    eh, ew = rh + dh, rw + dw_
                    tap = w[:, :, _KTAP[rh][dh], _KTAP[rw][dw_]]
                    wb = wb.at[(eh * 3 + ew) * cin:(eh * 3 + ew + 1) * cin,
                               (rh * 2 + rw) * cout:(rh * 2 + rw + 1) * cout
                               ].set(tap)
    return wb


def _patchmm_kernel(xp_ref, w_ref, b_ref, out_ref, *, rb, wdim, cin, nout, act):
    s = pl.program_id(1)
    base = s * rb
    chunks = []
    for eh in range(3):
        for ew in range(3):
            sl = xp_ref[0, pl.ds(base + eh, rb), ew:ew + wdim, :]
            chunks.append(sl.reshape(rb * wdim, cin))
    p = jnp.concatenate(chunks, axis=1)
    res = jax.lax.dot_general(p, w_ref[...], (((1,), (0,)), ((), ())),
                              preferred_element_type=jnp.float32)
    res = res + b_ref[...]
    if act == "relu":
        res = jnp.maximum(res, 0.0)
    else:
        res = 1.0 / (1.0 + jnp.exp(-res))
    out_ref[0] = res[:, :nout].reshape(rb, wdim, nout)


@functools.partial(jax.jit, static_argnames=("rb", "nout", "act"))
def _patchmm(xh, wb, bb, rb, nout, act):
    bsz, h, wdim, cin = xh.shape
    nmat = wb.shape[1]
    xp = jnp.pad(xh, ((0, 0), (1, 1), (1, 1), (0, 0)))
    nstrip = h // rb
    body = functools.partial(_patchmm_kernel, rb=rb, wdim=wdim, cin=cin,
                             nout=nout, act=act)
    return pl.pallas_call(
        body,
        grid=(bsz, nstrip),
        in_specs=[
            pl.BlockSpec((1, h + 2, wdim + 2, cin), lambda i, s: (i, 0, 0, 0)),
            pl.BlockSpec((9 * cin, nmat), lambda i, s: (0, 0)),
            pl.BlockSpec((1, nmat), lambda i, s: (0, 0)),
        ],
        out_specs=pl.BlockSpec((1, rb, wdim, nout),
                               lambda i, s: (i, s, 0, 0)),
        out_shape=jax.ShapeDtypeStruct((bsz, h, wdim, nout), jnp.float32),
    )(xp, wb, bb)


def _deconv_phase(xh, w, b, rb):
    cout = w.shape[1]
    wb = _phase_weight(w)
    bb = jnp.tile(b, 4)[None, :]
    return _patchmm(xh, wb, bb, rb, 4 * cout, "relu")


def _interleave(ph, cout):
    bsz, h, wdim = ph.shape[:3]
    ph = ph.reshape(bsz, h, wdim, 2, 2, cout)
    ph = ph.transpose(0, 1, 3, 2, 4, 5)
    return ph.reshape(bsz, 2 * h, 2 * wdim, cout)


def _conv3_kernel(a0_ref, a1_ref, a2_ref, w_ref, b_ref, out_ref,
                  *, rb, nsc):
    refs = (a0_ref, a1_ref, a2_ref)
    chunks = []
    for eh in range(3):
        for rs in range(3):
            sl = refs[eh][0, :, rs:rs + nsc, :]
            chunks.append(sl.reshape(rb * nsc, 128))
    p = jnp.concatenate(chunks, axis=1)
    res = jax.lax.dot_general(p, w_ref[...], (((1,), (0,)), ((), ())),
                              preferred_element_type=jnp.float32)
    res = 1.0 / (1.0 + jnp.exp(-(res + b_ref[...])))
    out_ref[0] = res.reshape(rb, nsc, 128)


@functools.partial(jax.jit, static_argnames=("rb",))
def _conv3(xh, wb, bb, rb):
    bsz, h, wdim, cin = xh.shape
    nsc = wdim // 4
    xp = jnp.pad(xh, ((0, 0), (1, 1), (4, 4), (0, 0)))
    g = xp.reshape(bsz, h + 2, nsc + 2, 4 * cin)
    shifted = [g[:, eh:eh + h] for eh in range(3)]
    nstrip = h // rb
    body = functools.partial(_conv3_kernel, rb=rb, nsc=nsc)
    inspec = pl.BlockSpec((1, rb, nsc + 2, 128), lambda i, s: (i, s, 0, 0))
    return pl.pallas_call(
        body,
        grid=(bsz, nstrip),
        in_specs=[
            inspec, inspec, inspec,
            pl.BlockSpec((1152, 128), lambda i, s: (0, 0)),
            pl.BlockSpec((1, 128), lambda i, s: (0, 0)),
        ],
        out_specs=pl.BlockSpec((1, rb, nsc, 128), lambda i, s: (i, s, 0, 0)),
        out_shape=jax.ShapeDtypeStruct((bsz, h, nsc, 128), jnp.float32),
    )(*shifted, wb, bb)


def kernel(x, w1, b1, w2, b2, emb, dw1, db1, dw2, db2, w3, b3):
    z = jax.nn.relu(_conv2d(x, w1, b1, 2, 1))
    z = jax.nn.relu(_conv2d(z, w2, b2, 2, 1))
    B, C, H, W = z.shape
    flat = z.transpose(0, 2, 3, 1).reshape(-1, C)
    q_flat, sq = _vq(flat, emb)
    loss = 1.25 * sq / (flat.shape[0] * C)
    q_nhwc = q_flat.reshape(B, H, W, C)

    h1 = _interleave(_deconv_phase(q_nhwc, dw1, db1, rb=56), dw1.shape[1])
    h2 = _interleave(_deconv_phase(h1, dw2, db2, rb=56), dw2.shape[1])

    cin3, co3 = w3.shape[1], w3.shape[0]
    w3f = jnp.zeros((1152, 128), w3.dtype)
    for eh in range(3):
        for dw_ in range(3):
            tap = w3[:, :, eh, dw_].T
            for j4o in range(4):
                rs, j4i = (j4o + dw_ + 3) // 4, (j4o + dw_ + 3) % 4
                r0 = (eh * 3 + rs) * 128 + j4i * 32
                w3f = w3f.at[r0:r0 + cin3, j4o * 32:j4o * 32 + co3].set(tap)
    b3f = jnp.zeros((1, 128), b3.dtype)
    for j4o in range(4):
        b3f = b3f.at[0, j4o * 32:j4o * 32 + co3].set(b3)
    rec = _conv3(h2, w3f, b3f, rb=56)
    recon_nhwc = rec.reshape(8, 224, 224, 32)[..., :co3]
    recon = recon_nhwc.transpose(0, 3, 1, 2)
    q_st = q_nhwc.transpose(0, 3, 1, 2)
    return (recon, loss, q_st)
